# Initial kernel scaffold; baseline (speedup 1.0000x reference)
import jax
import jax.numpy as jnp
from jax.experimental import pallas as pl


def kernel(x, indices, weights):
    raise NotImplementedError("write your pallas kernel here")


# speedup vs baseline: 16.2215x; 16.2215x over previous
import functools

import jax
import jax.numpy as jnp
from jax import lax
from jax.experimental import pallas as pl
from jax.experimental.pallas import tpu as pltpu
from jax.experimental.pallas import tpu_sc as plsc

N = 16384
COLS = 64
HCOLS = COLS // 2
NC = 2
NS = 16
K = 1024
ROWS_PER_TILE = N // NS


def _spmm2_kernel(nnz_pad):
    e_tile = nnz_pad // NS
    n_chunks = e_tile // K
    mesh = plsc.VectorSubcoreMesh(
        core_axis_name="c", subcore_axis_name="s",
        num_cores=NC, num_subcores=NS)

    @functools.partial(
        pl.kernel,
        out_type=(
            jax.ShapeDtypeStruct((NC * N, HCOLS), jnp.float32),
            jax.ShapeDtypeStruct((NC * N, HCOLS), jnp.float32),
        ),
        mesh=mesh,
        compiler_params=pltpu.CompilerParams(use_tc_tiling_on_sc=False),
        scratch_types=[
            pltpu.VMEM_SHARED((N, HCOLS), jnp.float32),
            pltpu.VMEM((K,), jnp.int32),
            pltpu.VMEM((K,), jnp.int32),
            pltpu.VMEM((K,), jnp.float32),
            pltpu.VMEM((K, HCOLS), jnp.float32),
            pltpu.SemaphoreType.DMA,
        ],
    )
    def k(xh, col_both, rowi, wts, zrows, out, x1h, acc, cidx, ridx, wv, gv, sem):
        h = lax.axis_index("c")
        sid = lax.axis_index("s")
        row_base = sid * ROWS_PER_TILE

        def edge_loop(src_hbm):
            def chunk_body(c, _):
                start = sid * e_tile + c * K
                pltpu.sync_copy(col_both.at[h, pl.ds(start, K)], cidx)
                pltpu.sync_copy(rowi.at[pl.ds(start, K)], ridx)
                pltpu.sync_copy(wts.at[pl.ds(start, K)], wv)
                pltpu.async_copy(src_hbm.at[cidx], gv, sem).wait()

                def scale_body(g, _):
                    base = g * 16
                    w16 = wv[pl.ds(base, 16)]
                    for r in range(16):
                        i = base + r
                        w = w16[r]
                        gv[i, pl.ds(0, 16)] = gv[i, pl.ds(0, 16)] * w
                        gv[i, pl.ds(16, 16)] = gv[i, pl.ds(16, 16)] * w
                    return 0

                lax.fori_loop(0, K // 16, scale_body, 0, unroll=False)
                pltpu.sync_copy(gv, acc.at[ridx], add=True)
                return 0

            lax.fori_loop(0, n_chunks, chunk_body, 0, unroll=False)

        def dump_acc(dst_hbm):
            pltpu.sync_copy(
                acc.at[pl.ds(row_base, ROWS_PER_TILE)],
                dst_hbm.at[pl.ds(h * N + row_base, ROWS_PER_TILE)])

        pltpu.sync_copy(zrows, acc.at[pl.ds(row_base, ROWS_PER_TILE)])
        plsc.subcore_barrier()
        edge_loop(xh)
        plsc.subcore_barrier()
        dump_acc(x1h)
        pltpu.sync_copy(zrows, acc.at[pl.ds(row_base, ROWS_PER_TILE)])
        plsc.subcore_barrier()
        edge_loop(x1h)
        plsc.subcore_barrier()
        dump_acc(out)

    return k


def kernel(x, indices, weights):
    nnz = weights.shape[0]
    chunk_all = NS * K
    nnz_pad = ((nnz + chunk_all - 1) // chunk_all) * chunk_all
    pad = nnz_pad - nnz

    row = indices[0]
    col = indices[1]
    if pad:
        row = jnp.pad(row, (0, pad))
        col = jnp.pad(col, (0, pad))
        weights = jnp.pad(weights, (0, pad))
    col_both = jnp.stack([col, col + N], axis=0)

    xh = jnp.concatenate([x[:, :HCOLS], x[:, HCOLS:]], axis=0)
    zrows = jnp.zeros((ROWS_PER_TILE, HCOLS), jnp.float32)

    out_h, _ = _spmm2_kernel(nnz_pad)(xh, col_both, row, weights, zrows)
    return jnp.concatenate([out_h[:N], out_h[N:]], axis=1)
